# Initial kernel scaffold; baseline (speedup 1.0000x reference)
import jax
import jax.numpy as jnp
from jax.experimental import pallas as pl


def kernel(x, A_indices, A_values, W1, b1, W2, b2):
    raise NotImplementedError("write your pallas kernel here")


# speedup vs baseline: 2.4613x; 2.4613x over previous
import functools
import jax
import jax.numpy as jnp
from jax import lax
from jax.experimental import pallas as pl
from jax.experimental.pallas import tpu as pltpu
from jax.experimental.pallas import tpu_sc as plsc

N_NODES = 10000
N_EDGES = 320000
D_IN = 128
D_HID = 256
D_OUT = 128

NC = 2
NS = 16
NW = NC * NS
CHUNK = 128
CPW = 80
EPAD = NW * CPW * CHUNK
ROWS_PER_TILE = 624
MROW_BLK = 1000

_sc_mesh = plsc.VectorSubcoreMesh(
    core_axis_name="c", subcore_axis_name="s", num_cores=NC, num_subcores=NS)


@functools.partial(
    pl.kernel,
    out_type=jax.ShapeDtypeStruct((NC, N_NODES, 128), jnp.float32),
    mesh=_sc_mesh,
    scratch_types=[
        pltpu.VMEM((CPW, CHUNK), jnp.int32),
        pltpu.VMEM((CPW, CHUNK), jnp.int32),
        pltpu.VMEM((CPW, CHUNK), jnp.float32),
        pltpu.VMEM((CHUNK, 128), jnp.float32),
        pltpu.VMEM_SHARED((N_NODES, 128), jnp.float32),
        pltpu.SemaphoreType.DMA,
    ],
)
def _spmm_sc(row_hbm, col_hbm, val_hbm, h_hbm, out_hbm,
             ridx, cidx, vals, gbuf, acc, sem):
    c = lax.axis_index("c")
    s = lax.axis_index("s")
    wid = c * NS + s

    pltpu.sync_copy(row_hbm.at[wid], ridx)
    pltpu.sync_copy(col_hbm.at[wid], cidx)
    pltpu.sync_copy(val_hbm.at[wid], vals)

    zero = jnp.zeros((16,), jnp.float32)

    def zrow(r, carry):
        for d in range(8):
            gbuf[r, pl.ds(d * 16, 16)] = zero
        return carry

    lax.fori_loop(0, CHUNK, zrow, 0)
    rbase = s * ROWS_PER_TILE
    for k in range(4):
        pltpu.sync_copy(gbuf, acc.at[pl.ds(rbase + k * 128, 128)])
    pltpu.sync_copy(gbuf.at[pl.ds(0, ROWS_PER_TILE - 512)],
                    acc.at[pl.ds(rbase + 512, ROWS_PER_TILE - 512)])

    @pl.when(s == NS - 1)
    def _zero_tail():
        pltpu.sync_copy(gbuf.at[pl.ds(0, N_NODES - NS * ROWS_PER_TILE)],
                        acc.at[pl.ds(NS * ROWS_PER_TILE,
                                     N_NODES - NS * ROWS_PER_TILE)])

    plsc.subcore_barrier()

    def chunk_body(j, carry):
        pltpu.async_copy(h_hbm.at[cidx.at[j]], gbuf, sem).wait()

        def escale(k, ecarry):
            vv = vals[j, pl.ds(k * 16, 16)]
            for i in range(16):
                v = vv[i]
                e = k * 16 + i
                for d in range(8):
                    sl = pl.ds(d * 16, 16)
                    gbuf[e, sl] = gbuf[e, sl] * v
            return ecarry

        lax.fori_loop(0, CHUNK // 16, escale, 0)

        pltpu.sync_copy(gbuf, acc.at[ridx.at[j]], add=True)
        return carry

    lax.fori_loop(0, CPW, chunk_body, 0)
    plsc.subcore_barrier()

    pltpu.sync_copy(acc.at[pl.ds(rbase, ROWS_PER_TILE)],
                    out_hbm.at[c, pl.ds(rbase, ROWS_PER_TILE)])

    @pl.when(s == NS - 1)
    def _flush_tail():
        pltpu.sync_copy(
            acc.at[pl.ds(NS * ROWS_PER_TILE, N_NODES - NS * ROWS_PER_TILE)],
            out_hbm.at[c, pl.ds(NS * ROWS_PER_TILE,
                                N_NODES - NS * ROWS_PER_TILE)])


def _dense1_body(x_ref, w_ref, b_ref, ha_ref, hb_ref):
    h = jnp.dot(x_ref[...], w_ref[...],
                preferred_element_type=jnp.float32) + b_ref[...]
    ha_ref[...] = h[:, :128]
    hb_ref[...] = h[:, 128:]


def _dense2_body(pa_ref, pb_ref, wa_ref, wb_ref, b_ref, out_ref):
    s0 = jnp.maximum(pa_ref[0] + pa_ref[1], 0.0)
    s1 = jnp.maximum(pb_ref[0] + pb_ref[1], 0.0)
    out_ref[...] = (
        jnp.dot(s0, wa_ref[...], preferred_element_type=jnp.float32)
        + jnp.dot(s1, wb_ref[...], preferred_element_type=jnp.float32)
        + b_ref[...])


def _softmax_body(p_ref, out_ref):
    z = p_ref[0] + p_ref[1]
    m = jnp.max(z, axis=1, keepdims=True)
    e = jnp.exp(z - m)
    out_ref[...] = e / jnp.sum(e, axis=1, keepdims=True)


_GRID = N_NODES // MROW_BLK

_dense1 = pl.pallas_call(
    _dense1_body,
    grid=(_GRID,),
    in_specs=[
        pl.BlockSpec((MROW_BLK, D_IN), lambda i: (i, 0)),
        pl.BlockSpec((D_IN, D_HID), lambda i: (0, 0)),
        pl.BlockSpec((1, D_HID), lambda i: (0, 0)),
    ],
    out_specs=[
        pl.BlockSpec((MROW_BLK, 128), lambda i: (i, 0)),
        pl.BlockSpec((MROW_BLK, 128), lambda i: (i, 0)),
    ],
    out_shape=[
        jax.ShapeDtypeStruct((N_NODES, 128), jnp.float32),
        jax.ShapeDtypeStruct((N_NODES, 128), jnp.float32),
    ],
)

_dense2 = pl.pallas_call(
    _dense2_body,
    grid=(_GRID,),
    in_specs=[
        pl.BlockSpec((NC, MROW_BLK, 128), lambda i: (0, i, 0)),
        pl.BlockSpec((NC, MROW_BLK, 128), lambda i: (0, i, 0)),
        pl.BlockSpec((128, D_OUT), lambda i: (0, 0)),
        pl.BlockSpec((128, D_OUT), lambda i: (0, 0)),
        pl.BlockSpec((1, D_OUT), lambda i: (0, 0)),
    ],
    out_specs=pl.BlockSpec((MROW_BLK, D_OUT), lambda i: (i, 0)),
    out_shape=jax.ShapeDtypeStruct((N_NODES, D_OUT), jnp.float32),
)

_softmax = pl.pallas_call(
    _softmax_body,
    grid=(_GRID,),
    in_specs=[pl.BlockSpec((NC, MROW_BLK, D_OUT), lambda i: (0, i, 0))],
    out_specs=pl.BlockSpec((MROW_BLK, D_OUT), lambda i: (i, 0)),
    out_shape=jax.ShapeDtypeStruct((N_NODES, D_OUT), jnp.float32),
)


def kernel(x, A_indices, A_values, W1, b1, W2, b2):
    pad = EPAD - N_EDGES
    row = jnp.concatenate(
        [A_indices[0].astype(jnp.int32), jnp.zeros((pad,), jnp.int32)])
    col = jnp.concatenate(
        [A_indices[1].astype(jnp.int32), jnp.zeros((pad,), jnp.int32)])
    val = jnp.concatenate(
        [A_values.astype(jnp.float32), jnp.zeros((pad,), jnp.float32)])
    row3 = row.reshape(NW, CPW, CHUNK)
    col3 = col.reshape(NW, CPW, CHUNK)
    val3 = val.reshape(NW, CPW, CHUNK)

    ha, hb = _dense1(x, W1, b1.reshape(1, D_HID))
    p1a = _spmm_sc(row3, col3, val3, ha)
    p1b = _spmm_sc(row3, col3, val3, hb)
    h2 = _dense2(p1a, p1b, W2[:128], W2[128:], b2.reshape(1, D_OUT))
    p2 = _spmm_sc(row3, col3, val3, h2)
    return _softmax(p2)


# speedup vs baseline: 2.9321x; 1.1913x over previous
import functools
import jax
import jax.numpy as jnp
from jax import lax
from jax.experimental import pallas as pl
from jax.experimental.pallas import tpu as pltpu
from jax.experimental.pallas import tpu_sc as plsc

N_NODES = 10000
N_EDGES = 320000
D_IN = 128
D_HID = 256
D_OUT = 128

NC = 2
NS = 16
NW = NC * NS
CHUNK = 128
CPW = 80
NBUF = 2
EPAD = NW * CPW * CHUNK
ROWS_PER_TILE = 624
MROW_BLK = 1000

_sc_mesh = plsc.VectorSubcoreMesh(
    core_axis_name="c", subcore_axis_name="s", num_cores=NC, num_subcores=NS)


@functools.partial(
    pl.kernel,
    out_type=jax.ShapeDtypeStruct((NC, N_NODES, 128), jnp.float32),
    mesh=_sc_mesh,
    scratch_types=[
        pltpu.VMEM((NBUF, CHUNK), jnp.int32),
        pltpu.VMEM((CPW, CHUNK), jnp.int32),
        pltpu.VMEM((NBUF, CHUNK), jnp.float32),
        [pltpu.VMEM((CHUNK, 128), jnp.float32) for _ in range(NBUF)],
        pltpu.VMEM_SHARED((N_NODES, 128), jnp.float32),
        [pltpu.SemaphoreType.DMA for _ in range(NBUF)],
    ],
)
def _spmm_sc(row_hbm, col_hbm, val_hbm, h_hbm, out_hbm,
             ridx, cidx, vals, gbufs, acc, gsems):
    gbuf = gbufs[0]
    c = lax.axis_index("c")
    s = lax.axis_index("s")
    wid = c * NS + s

    pltpu.sync_copy(col_hbm.at[wid], cidx)

    zero = jnp.zeros((16,), jnp.float32)

    def zrow(r, carry):
        for d in range(8):
            gbuf[r, pl.ds(d * 16, 16)] = zero
        return carry

    lax.fori_loop(0, CHUNK, zrow, 0)
    rbase = s * ROWS_PER_TILE
    for k in range(4):
        pltpu.sync_copy(gbuf, acc.at[pl.ds(rbase + k * 128, 128)])
    pltpu.sync_copy(gbuf.at[pl.ds(0, ROWS_PER_TILE - 512)],
                    acc.at[pl.ds(rbase + 512, ROWS_PER_TILE - 512)])

    @pl.when(s == NS - 1)
    def _zero_tail():
        pltpu.sync_copy(gbuf.at[pl.ds(0, N_NODES - NS * ROWS_PER_TILE)],
                        acc.at[pl.ds(NS * ROWS_PER_TILE,
                                     N_NODES - NS * ROWS_PER_TILE)])

    plsc.subcore_barrier()

    def issue(j, b):
        pltpu.async_copy(h_hbm.at[cidx.at[j]], gbufs[b], gsems[b])
        pltpu.async_copy(row_hbm.at[wid, j], ridx.at[b], gsems[b])
        pltpu.async_copy(val_hbm.at[wid, j], vals.at[b], gsems[b])

    def drain(j, b):
        pltpu.make_async_copy(h_hbm.at[cidx.at[j]], gbufs[b], gsems[b]).wait()
        pltpu.make_async_copy(row_hbm.at[wid, j], ridx.at[b], gsems[b]).wait()
        pltpu.make_async_copy(val_hbm.at[wid, j], vals.at[b], gsems[b]).wait()

    for b in range(NBUF):
        issue(b, b)

    n_outer = CPW // NBUF

    def outer_body(g, carry):
        for b in range(NBUF):
            j = g * NBUF + b
            buf = gbufs[b]
            drain(j, b)

            def escale(k, ecarry, buf=buf, b=b):
                vv = vals[b, pl.ds(k * 16, 16)]
                for i in range(16):
                    v = vv[i]
                    e = k * 16 + i
                    for d in range(8):
                        sl = pl.ds(d * 16, 16)
                        buf[e, sl] = buf[e, sl] * v
                return ecarry

            lax.fori_loop(0, CHUNK // 16, escale, 0)

            pltpu.sync_copy(buf, acc.at[ridx.at[b]], add=True)

            @pl.when(g < n_outer - 1)
            def _refill(b=b, j=j):
                issue(j + NBUF, b)

        return carry

    lax.fori_loop(0, n_outer, outer_body, 0)
    plsc.subcore_barrier()

    pltpu.sync_copy(acc.at[pl.ds(rbase, ROWS_PER_TILE)],
                    out_hbm.at[c, pl.ds(rbase, ROWS_PER_TILE)])

    @pl.when(s == NS - 1)
    def _flush_tail():
        pltpu.sync_copy(
            acc.at[pl.ds(NS * ROWS_PER_TILE, N_NODES - NS * ROWS_PER_TILE)],
            out_hbm.at[c, pl.ds(NS * ROWS_PER_TILE,
                                N_NODES - NS * ROWS_PER_TILE)])


def _dense1_body(x_ref, w_ref, b_ref, ha_ref, hb_ref):
    h = jnp.dot(x_ref[...], w_ref[...],
                preferred_element_type=jnp.float32) + b_ref[...]
    ha_ref[...] = h[:, :128]
    hb_ref[...] = h[:, 128:]


def _dense2_body(pa_ref, pb_ref, wa_ref, wb_ref, b_ref, out_ref):
    s0 = jnp.maximum(pa_ref[0] + pa_ref[1], 0.0)
    s1 = jnp.maximum(pb_ref[0] + pb_ref[1], 0.0)
    out_ref[...] = (
        jnp.dot(s0, wa_ref[...], preferred_element_type=jnp.float32)
        + jnp.dot(s1, wb_ref[...], preferred_element_type=jnp.float32)
        + b_ref[...])


def _softmax_body(p_ref, out_ref):
    z = p_ref[0] + p_ref[1]
    m = jnp.max(z, axis=1, keepdims=True)
    e = jnp.exp(z - m)
    out_ref[...] = e / jnp.sum(e, axis=1, keepdims=True)


_GRID = N_NODES // MROW_BLK

_dense1 = pl.pallas_call(
    _dense1_body,
    grid=(_GRID,),
    in_specs=[
        pl.BlockSpec((MROW_BLK, D_IN), lambda i: (i, 0)),
        pl.BlockSpec((D_IN, D_HID), lambda i: (0, 0)),
        pl.BlockSpec((1, D_HID), lambda i: (0, 0)),
    ],
    out_specs=[
        pl.BlockSpec((MROW_BLK, 128), lambda i: (i, 0)),
        pl.BlockSpec((MROW_BLK, 128), lambda i: (i, 0)),
    ],
    out_shape=[
        jax.ShapeDtypeStruct((N_NODES, 128), jnp.float32),
        jax.ShapeDtypeStruct((N_NODES, 128), jnp.float32),
    ],
)

_dense2 = pl.pallas_call(
    _dense2_body,
    grid=(_GRID,),
    in_specs=[
        pl.BlockSpec((NC, MROW_BLK, 128), lambda i: (0, i, 0)),
        pl.BlockSpec((NC, MROW_BLK, 128), lambda i: (0, i, 0)),
        pl.BlockSpec((128, D_OUT), lambda i: (0, 0)),
        pl.BlockSpec((128, D_OUT), lambda i: (0, 0)),
        pl.BlockSpec((1, D_OUT), lambda i: (0, 0)),
    ],
    out_specs=pl.BlockSpec((MROW_BLK, D_OUT), lambda i: (i, 0)),
    out_shape=jax.ShapeDtypeStruct((N_NODES, D_OUT), jnp.float32),
)

_softmax = pl.pallas_call(
    _softmax_body,
    grid=(_GRID,),
    in_specs=[pl.BlockSpec((NC, MROW_BLK, D_OUT), lambda i: (0, i, 0))],
    out_specs=pl.BlockSpec((MROW_BLK, D_OUT), lambda i: (i, 0)),
    out_shape=jax.ShapeDtypeStruct((N_NODES, D_OUT), jnp.float32),
)


def kernel(x, A_indices, A_values, W1, b1, W2, b2):
    pad = EPAD - N_EDGES
    row = jnp.concatenate(
        [A_indices[0].astype(jnp.int32), jnp.zeros((pad,), jnp.int32)])
    col = jnp.concatenate(
        [A_indices[1].astype(jnp.int32), jnp.zeros((pad,), jnp.int32)])
    val = jnp.concatenate(
        [A_values.astype(jnp.float32), jnp.zeros((pad,), jnp.float32)])
    row3 = row.reshape(NW, CPW, CHUNK)
    col3 = col.reshape(NW, CPW, CHUNK)
    val3 = val.reshape(NW, CPW, CHUNK)

    ha, hb = _dense1(x, W1, b1.reshape(1, D_HID))
    p1a = _spmm_sc(row3, col3, val3, ha)
    p1b = _spmm_sc(row3, col3, val3, hb)
    h2 = _dense2(p1a, p1b, W2[:128], W2[128:], b2.reshape(1, D_OUT))
    p2 = _spmm_sc(row3, col3, val3, h2)
    return _softmax(p2)
